# Initial kernel scaffold; baseline (speedup 1.0000x reference)
import jax
import jax.numpy as jnp
from jax.experimental import pallas as pl


def kernel(xs, mis_mask, x_proj_weight, w_route, b_route, w_noise, b_noise, noise_eps):
    raise NotImplementedError("write your pallas kernel here")


# speedup vs baseline: 1.0174x; 1.0174x over previous
import jax
import jax.numpy as jnp
from jax import lax
from jax.experimental import pallas as pl
from jax.experimental.pallas import tpu as pltpu

B, K, D, L = 4, 4, 1024, 2048
E, DT = 16, 16
LB = 512
NL = L // LB


def _mean_router_body(x_ref, wr_ref, br_ref, wn_ref, bn_ref, eps_ref,
                      mis_ref, gate_ref, zloss_ref, acc_ref, zacc_ref):
    b = pl.program_id(0)
    li = pl.program_id(1)

    x = x_ref[0]
    part = jnp.sum(jnp.sum(x, axis=2), axis=0)[None, :]

    @pl.when(li == 0)
    def _():
        acc_ref[...] = part

    @pl.when(li != 0)
    def _():
        acc_ref[...] += part

    @pl.when(li == NL - 1)
    def _():
        mean = acc_ref[...] * (1.0 / (K * L))
        route = lax.dot_general(mean, wr_ref[...],
                                (((1,), (1,)), ((), ())),
                                preferred_element_type=jnp.float32) + br_ref[...]
        noise_l = lax.dot_general(mean, wn_ref[...],
                                  (((1,), (1,)), ((), ())),
                                  preferred_element_type=jnp.float32) + bn_ref[...]
        logits = jax.nn.softmax(route, axis=-1)
        sp = jax.nn.softplus(noise_l)
        noise = jax.nn.softmax(eps_ref[0] * sp, axis=-1)
        noisy = logits + noise
        v = noisy[0]
        ij = lax.broadcasted_iota(jnp.int32, (E, E), 0)
        ie = lax.broadcasted_iota(jnp.int32, (E, E), 1)
        hit = (v[:, None] > v[None, :]) | ((v[:, None] == v[None, :]) & (ij < ie))
        rank = jnp.sum(hit.astype(jnp.int32), axis=0)
        sparse = jnp.where(rank < mis_ref[0, 0], v, 0.0)
        ex = jnp.exp(sparse - jnp.max(sparse))
        gate_ref[0] = (ex / jnp.sum(ex))[None, :]
        zterm = jnp.log(jnp.sum(jnp.exp(noisy))) ** 2

        @pl.when(b == 0)
        def _():
            zacc_ref[0, 0] = zterm

        @pl.when(b != 0)
        def _():
            zacc_ref[0, 0] += zterm

        @pl.when(b == B - 1)
        def _():
            zloss_ref[0, 0] = zacc_ref[0, 0] * (1.0 / B)


def _stage_a(xs, w_route, b_route2, w_noise, b_noise2, noise_eps3, mis3):
    return pl.pallas_call(
        _mean_router_body,
        grid=(B, NL),
        in_specs=[
            pl.BlockSpec((1, K, D, LB), lambda b, li: (b, 0, 0, li)),
            pl.BlockSpec((E, D), lambda b, li: (0, 0)),
            pl.BlockSpec((1, E), lambda b, li: (0, 0)),
            pl.BlockSpec((E, D), lambda b, li: (0, 0)),
            pl.BlockSpec((1, E), lambda b, li: (0, 0)),
            pl.BlockSpec((1, 1, E), lambda b, li: (b, 0, 0)),
            pl.BlockSpec((1, 1, E), lambda b, li: (b, 0, 0)),
        ],
        out_specs=[
            pl.BlockSpec((1, 1, E), lambda b, li: (b, 0, 0)),
            pl.BlockSpec((1, 1), lambda b, li: (0, 0),
                         memory_space=pltpu.SMEM),
        ],
        out_shape=[
            jax.ShapeDtypeStruct((B, 1, E), jnp.float32),
            jax.ShapeDtypeStruct((1, 1), jnp.float32),
        ],
        scratch_shapes=[
            pltpu.VMEM((1, D), jnp.float32),
            pltpu.SMEM((1, 1), jnp.float32),
        ],
    )(xs, w_route, b_route2, w_noise, b_noise2, noise_eps3, mis3)


def _proj_body(gate_ref, w_ref, x_ref, out_ref):
    g = gate_ref[0, 0]
    w = w_ref[:, 0]
    weff = jnp.sum(g[:, None, None] * w, axis=0)
    x = x_ref[0, 0]
    out_ref[0, 0] = lax.dot_general(weff, x, (((1,), (0,)), ((), ())),
                                    preferred_element_type=jnp.float32)


def _stage_c(gate3, x_proj_weight, xs):
    return pl.pallas_call(
        _proj_body,
        grid=(B, K),
        in_specs=[
            pl.BlockSpec((1, 1, E), lambda b, k: (b, 0, 0)),
            pl.BlockSpec((E, 1, DT, D), lambda b, k: (0, k, 0, 0)),
            pl.BlockSpec((1, 1, D, L), lambda b, k: (b, k, 0, 0)),
        ],
        out_specs=pl.BlockSpec((1, 1, DT, L), lambda b, k: (b, k, 0, 0)),
        out_shape=jax.ShapeDtypeStruct((B, K, DT, L), jnp.float32),
    )(gate3, x_proj_weight, xs)


def kernel(xs, mis_mask, x_proj_weight, w_route, b_route, w_noise, b_noise,
           noise_eps):
    b_route2 = b_route.reshape(1, E)
    b_noise2 = b_noise.reshape(1, E)
    mis3 = jnp.broadcast_to(mis_mask[:, None], (B, E)).astype(jnp.int32)
    mis3 = mis3.reshape(B, 1, E)

    gate3, zloss = _stage_a(xs, w_route, b_route2, w_noise, b_noise2,
                            noise_eps.reshape(B, 1, E), mis3)
    final_output = _stage_c(gate3, x_proj_weight, xs)
    return final_output, zloss[0, 0]


# speedup vs baseline: 1.5563x; 1.5298x over previous
import jax
import jax.numpy as jnp
from jax import lax
from jax.experimental import pallas as pl
from jax.experimental.pallas import tpu as pltpu

B, K, D, L = 4, 4, 1024, 2048
E, DT = 16, 16
NS = 5


def _rolling_body(x_ref, w_ref, wr_ref, br_ref, wn_ref, bn_ref, eps_ref,
                  mis_ref, out_ref, zloss_ref, cache_ref, gate_ref, acc_ref,
                  zacc_ref, sem):
    s = pl.program_id(0)
    k = pl.program_id(1)
    w_slot = lax.rem(4 * s + k, NS)
    r_slot = lax.rem(4 * s + k + 1, NS)

    @pl.when(s < B)
    def _():
        pltpu.make_async_copy(x_ref.at[0, 0], cache_ref.at[w_slot], sem).start()
        part = jnp.sum(x_ref[0, 0], axis=1)[None, :]

        @pl.when(k == 0)
        def _():
            acc_ref[...] = part

        @pl.when(k != 0)
        def _():
            acc_ref[...] += part

    @pl.when(s > 0)
    def _():
        g = gate_ref[0]
        weff = jnp.sum(g[:, None, None] * w_ref[:, k], axis=0)
        out_ref[0, 0] = lax.dot_general(weff, cache_ref[r_slot],
                                        (((1,), (0,)), ((), ())),
                                        preferred_element_type=jnp.float32)

    @pl.when((s < B) & (k == K - 1))
    def _():
        mean = acc_ref[...] * (1.0 / (K * L))
        route = lax.dot_general(mean, wr_ref[...],
                                (((1,), (1,)), ((), ())),
                                preferred_element_type=jnp.float32) + br_ref[...]
        noise_l = lax.dot_general(mean, wn_ref[...],
                                  (((1,), (1,)), ((), ())),
                                  preferred_element_type=jnp.float32) + bn_ref[...]
        logits = jax.nn.softmax(route, axis=-1)
        sp = jax.nn.softplus(noise_l)
        noise = jax.nn.softmax(eps_ref[0] * sp, axis=-1)
        noisy = logits + noise
        v = noisy[0]
        ij = lax.broadcasted_iota(jnp.int32, (E, E), 0)
        ie = lax.broadcasted_iota(jnp.int32, (E, E), 1)
        hit = (v[:, None] > v[None, :]) | ((v[:, None] == v[None, :]) & (ij < ie))
        rank = jnp.sum(hit.astype(jnp.int32), axis=0)
        sparse = jnp.where(rank < mis_ref[0, 0], v, 0.0)
        ex = jnp.exp(sparse - jnp.max(sparse))
        gate_ref[...] = (ex / jnp.sum(ex))[None, :]
        zterm = jnp.log(jnp.sum(jnp.exp(noisy))) ** 2

        @pl.when(s == 0)
        def _():
            zacc_ref[0, 0] = zterm

        @pl.when(s != 0)
        def _():
            zacc_ref[0, 0] += zterm

        @pl.when(s == B - 1)
        def _():
            zloss_ref[0, 0] = zacc_ref[0, 0] * (1.0 / B)

    @pl.when(s < B)
    def _():
        pltpu.make_async_copy(x_ref.at[0, 0], cache_ref.at[w_slot], sem).wait()


def kernel(xs, mis_mask, x_proj_weight, w_route, b_route, w_noise, b_noise,
           noise_eps):
    b_route2 = b_route.reshape(1, E)
    b_noise2 = b_noise.reshape(1, E)
    mis3 = jnp.broadcast_to(mis_mask[:, None], (B, E)).astype(jnp.int32)
    mis3 = mis3.reshape(B, 1, E)
    eps3 = noise_eps.reshape(B, 1, E)

    out, zloss = pl.pallas_call(
        _rolling_body,
        grid=(B + 1, K),
        in_specs=[
            pl.BlockSpec((1, 1, D, L),
                         lambda s, k: (jnp.where(s < B, s, B - 1),
                                       jnp.where(s < B, k, K - 1), 0, 0)),
            pl.BlockSpec((E, K, DT, D), lambda s, k: (0, 0, 0, 0)),
            pl.BlockSpec((E, D), lambda s, k: (0, 0)),
            pl.BlockSpec((1, E), lambda s, k: (0, 0)),
            pl.BlockSpec((E, D), lambda s, k: (0, 0)),
            pl.BlockSpec((1, E), lambda s, k: (0, 0)),
            pl.BlockSpec((1, 1, E), lambda s, k: (jnp.where(s < B, s, B - 1), 0, 0)),
            pl.BlockSpec((1, 1, E), lambda s, k: (jnp.where(s < B, s, B - 1), 0, 0)),
        ],
        out_specs=[
            pl.BlockSpec((1, 1, DT, L),
                         lambda s, k: (jnp.where(s > 0, s - 1, B), k, 0, 0)),
            pl.BlockSpec((1, 1), lambda s, k: (0, 0),
                         memory_space=pltpu.SMEM),
        ],
        out_shape=[
            jax.ShapeDtypeStruct((B + 1, K, DT, L), jnp.float32),
            jax.ShapeDtypeStruct((1, 1), jnp.float32),
        ],
        scratch_shapes=[
            pltpu.VMEM((NS, D, L), jnp.float32),
            pltpu.VMEM((1, E), jnp.float32),
            pltpu.VMEM((1, D), jnp.float32),
            pltpu.SMEM((1, 1), jnp.float32),
            pltpu.SemaphoreType.DMA,
        ],
        compiler_params=pltpu.CompilerParams(
            vmem_limit_bytes=63 * 1024 * 1024,
        ),
    )(xs, x_proj_weight, w_route, b_route2, w_noise, b_noise2, eps3, mis3)
    return out[:B], zloss[0, 0]
